# Initial kernel scaffold; baseline (speedup 1.0000x reference)
import jax
import jax.numpy as jnp
from jax.experimental import pallas as pl


def kernel(x, edge_index, batch, W1, b1, W2, b2):
    raise NotImplementedError("write your pallas kernel here")


# speedup vs baseline: 13.8207x; 13.8207x over previous
import functools

import jax
import jax.numpy as jnp
from jax import lax
from jax.experimental import pallas as pl
from jax.experimental.pallas import tpu as pltpu
from jax.experimental.pallas import tpu_sc as plsc

N = 10000
NPAD = 10112
D = 128
E = 320000
G = 16
NC = 2
NS = 16
NW = NC * NS
BLK = 128
NBLK = 79
EPAD = NW * NBLK * BLK
ROWS_PER_SUB = NPAD // NS

_sc_mesh = plsc.VectorSubcoreMesh(core_axis_name="c", subcore_axis_name="s")


@functools.partial(
    pl.kernel,
    out_type=jax.ShapeDtypeStruct((NW, NPAD), jnp.float32),
    mesh=_sc_mesh,
    scratch_types=[
        pltpu.VMEM((NBLK, BLK), jnp.int32),
        pltpu.VMEM((NPAD,), jnp.float32),
    ],
    compiler_params=pltpu.CompilerParams(needs_layout_passes=False),
)
def _deg_kernel(dst_hbm, out_hbm, idx_v, deg_v):
    cid = lax.axis_index("c")
    sid = lax.axis_index("s")
    wid = cid * NS + sid
    pltpu.sync_copy(dst_hbm.at[cid, sid], idx_v)

    zeros16 = jnp.zeros((16,), jnp.float32)
    ones16 = jnp.ones((16,), jnp.float32)

    def zero_body(i, _):
        deg_v[pl.ds(i * 16, 16)] = zeros16
        return ()

    lax.fori_loop(0, NPAD // 16, zero_body, ())

    def blk_body(j, _):
        def lane_body(k, _):
            idx = idx_v[j, pl.ds(k * 16, 16)]
            plsc.addupdate_scatter(deg_v, [idx], ones16)
            return ()

        lax.fori_loop(0, BLK // 16, lane_body, ())
        return ()

    lax.fori_loop(0, NBLK, blk_body, ())
    pltpu.sync_copy(deg_v, out_hbm.at[wid])


@functools.partial(
    pl.kernel,
    out_type=jax.ShapeDtypeStruct((NC, NPAD, D), jnp.float32),
    mesh=_sc_mesh,
    scratch_types=[
        pltpu.VMEM((NBLK, BLK), jnp.int32),
        pltpu.VMEM((NBLK, BLK), jnp.int32),
        pltpu.VMEM((BLK, D), jnp.float32),
        pltpu.VMEM_SHARED((NPAD, D), jnp.float32),
        pltpu.SemaphoreType.DMA,
    ],
)
def _agg_kernel(h_hbm, src_hbm, dst_hbm, zeros_hbm, out_hbm,
                src_v, dst_v, rows_v, acc_sh, sem):
    cid = lax.axis_index("c")
    sid = lax.axis_index("s")
    lo = sid * ROWS_PER_SUB

    pltpu.sync_copy(src_hbm.at[cid, sid], src_v)
    pltpu.sync_copy(dst_hbm.at[cid, sid], dst_v)
    pltpu.sync_copy(zeros_hbm.at[pl.ds(lo, ROWS_PER_SUB)],
                    acc_sh.at[pl.ds(lo, ROWS_PER_SUB)])
    plsc.subcore_barrier()

    def blk_body(j, _):
        pltpu.async_copy(h_hbm.at[src_v.at[j]], rows_v, sem).wait()
        pltpu.sync_copy(rows_v, acc_sh.at[dst_v.at[j]], add=True)
        return ()

    lax.fori_loop(0, NBLK, blk_body, ())
    plsc.subcore_barrier()
    pltpu.sync_copy(acc_sh.at[pl.ds(lo, ROWS_PER_SUB)],
                    out_hbm.at[cid, pl.ds(lo, ROWS_PER_SUB)])


def _prescale_body(degT_ref, x_ref, w_ref, dinv_ref, hp_ref):
    deg = jnp.sum(degT_ref[...], axis=1, keepdims=True) + 1.0
    dinv = lax.rsqrt(deg)[:N]
    h = jnp.dot(x_ref[...], w_ref[...], preferred_element_type=jnp.float32)
    dinv_ref[...] = dinv
    hp_ref[...] = dinv * h


def _mid_body(acc_ref, hp_ref, dinv_ref, b_ref, w_ref, out_ref):
    agg = acc_ref[0, :N] + acc_ref[1, :N] + hp_ref[...]
    dinv = dinv_ref[...]
    h = jnp.maximum(dinv * agg + b_ref[...], 0.0)
    out_ref[...] = dinv * jnp.dot(h, w_ref[...],
                                  preferred_element_type=jnp.float32)


def _final_body(acc_ref, hp_ref, dinv_ref, b_ref, batch_ref, out_ref):
    agg = acc_ref[0, :N] + acc_ref[1, :N] + hp_ref[...]
    h = jnp.maximum(dinv_ref[...] * agg + b_ref[...], 0.0)
    gids = lax.broadcasted_iota(jnp.int32, (G, N), 0)
    mask = (batch_ref[...] == gids).astype(jnp.float32)
    sums = jnp.dot(mask, h, preferred_element_type=jnp.float32)
    counts = jnp.sum(mask, axis=1, keepdims=True)
    pooled = sums / jnp.maximum(counts, 1.0)
    m = jnp.max(pooled, axis=1, keepdims=True)
    lse = jnp.log(jnp.sum(jnp.exp(pooled - m), axis=1, keepdims=True)) + m
    out_ref[...] = pooled - lse


_f32 = jnp.float32

_prescale = pl.pallas_call(
    _prescale_body,
    out_shape=[jax.ShapeDtypeStruct((N, 1), _f32),
               jax.ShapeDtypeStruct((N, D), _f32)],
)

_mid = pl.pallas_call(
    _mid_body,
    out_shape=jax.ShapeDtypeStruct((N, D), _f32),
)

_final = pl.pallas_call(
    _final_body,
    out_shape=jax.ShapeDtypeStruct((G, D), _f32),
)


def kernel(x, edge_index, batch, W1, b1, W2, b2):
    src = edge_index[0]
    dst = edge_index[1]
    pad = EPAD - E
    src4 = jnp.concatenate([src, jnp.zeros((pad,), jnp.int32)])
    src4 = src4.reshape(NC, NS, NBLK, BLK)
    dst4 = jnp.concatenate([dst, jnp.full((pad,), N, jnp.int32)])
    dst4 = dst4.reshape(NC, NS, NBLK, BLK)

    degP = _deg_kernel(dst4)
    degT = degP.T
    dinv, h1p = _prescale(degT, x, W1)

    zeros = jnp.zeros((NPAD, D), _f32)
    acc1 = _agg_kernel(h1p, src4, dst4, zeros)
    h2p = _mid(acc1, h1p, dinv, b1.reshape(1, D), W2)
    acc2 = _agg_kernel(h2p, src4, dst4, zeros)
    out = _final(acc2, h2p, dinv, b2.reshape(1, D), batch.reshape(1, N))
    return out
